# Initial kernel scaffold; baseline (speedup 1.0000x reference)
import jax
import jax.numpy as jnp
from jax.experimental import pallas as pl


def kernel(x, neighbor, W1, b1, Wt1, bt1, g1, be1, W2, b2, Wt2, bt2, g2, be2, Wc1, bc1, Wc2, bc2):
    raise NotImplementedError("write your pallas kernel here")


# speedup vs baseline: 2.7782x; 2.7782x over previous
import functools

import jax
import jax.numpy as jnp
from jax.experimental import pallas as pl
from jax.experimental.pallas import tpu as pltpu

_EPS_BN = 1e-5


def _fused_kernel(xs_ref, wt1t_ref, a1_ref, d1_ref, wt2_ref, w2st_ref,
                  b2s_ref, wc1t_ref, bc1_ref, wc2t_ref, bc2_ref, out_ref):
    xs = xs_ref[...]
    x = xs[:, 0, :]
    s = jnp.sum(xs[:, 1:, :], axis=1)

    raw = x[:, :, None] * s[:, None, :] + s[:, :, None] * x[:, None, :]
    q = jnp.sqrt(jnp.abs(raw))
    a_mat = jnp.where(raw < 0, -q, q)
    denom = jnp.sum(q, axis=1) + 1e-7

    e_all = jnp.einsum('nmf,nfg->nmg', xs, a_mat,
                       preferred_element_type=jnp.float32)
    e_all = e_all / denom[:, None, :]
    t_all = jnp.einsum('nmg,gh->nmh', e_all, wt1t_ref[...],
                       preferred_element_type=jnp.float32)

    a1 = a1_ref[...]
    d1 = d1_ref[...]
    pre = a1[0, None, :, None] * t_all[:, :, None, :] + d1[None, None]
    act = pre / (1.0 + jnp.abs(pre))
    khop = act[:, 0]
    snbr = jnp.sum(act[:, 1:], axis=1)

    raw2 = jnp.einsum('ncf,ncg->nfg', khop, snbr,
                      preferred_element_type=jnp.float32)
    raw2 = raw2 + jnp.transpose(raw2, (0, 2, 1))
    q2 = jnp.sqrt(jnp.abs(raw2))
    a2_mat = jnp.where(raw2 < 0, -q2, q2)
    denom2 = jnp.sum(q2, axis=1) + 1e-7

    e2 = jnp.einsum('ncf,nfg->ncg', khop, a2_mat,
                    preferred_element_type=jnp.float32)
    e2 = e2 / denom2[:, None, :]

    v = jnp.einsum('ncg,gz->ncz', e2, wt2_ref[...],
                   preferred_element_type=jnp.float32)[:, :, 0]
    z = jnp.dot(v, w2st_ref[...],
                preferred_element_type=jnp.float32) + b2s_ref[...]
    flat = z / (1.0 + jnp.abs(z))

    h = jnp.dot(flat, wc1t_ref[...],
                preferred_element_type=jnp.float32) + bc1_ref[...]
    h = jnp.maximum(h, 0.0)
    out_ref[...] = jnp.dot(h, wc2t_ref[...],
                           preferred_element_type=jnp.float32) + bc2_ref[...]


@jax.jit
def _kcat(x, neighbor, W1, b1, Wt1, bt1, g1, be1, W2, b2, Wt2, bt2, g2, be2,
          Wc1, bc1, Wc2, bc2):
    n = x.shape[0]
    f = x.shape[2]
    num_class = Wc2.shape[0]

    xs = jnp.concatenate([x, neighbor[:, 0, :, 0, :]], axis=1)

    scale1 = g1 / jnp.sqrt(1.0 + _EPS_BN)
    a1 = (scale1 * W1[:, 0]).reshape(1, -1)
    swt1 = jnp.sum(Wt1, axis=1)
    d1 = scale1[:, None] * (b1[:, None] * swt1[None, :] + bt1[None, :]) \
        + be1[:, None]

    scale2 = g2 / jnp.sqrt(1.0 + _EPS_BN)
    swt2 = jnp.sum(Wt2[0])
    w2st = (scale2[:, None] * W2).T
    b2s = (scale2 * (b2 * swt2 + bt2[0]) + be2).reshape(1, -1)

    wt1t = Wt1.T
    wt2 = Wt2.T
    wc1t = Wc1.T
    wc2t = Wc2.T
    bc1_2d = bc1.reshape(1, -1)
    bc2_2d = bc2.reshape(1, -1)

    for blk in (80, 40, 16, 8, 5, 4, 2, 1):
        if n % blk == 0:
            break

    full = lambda *shape: pl.BlockSpec(shape, lambda i: (0,) * len(shape))
    return pl.pallas_call(
        _fused_kernel,
        grid=(n // blk,),
        in_specs=[
            pl.BlockSpec((blk, xs.shape[1], f), lambda i: (i, 0, 0)),
            full(f, wt1t.shape[1]),
            full(1, a1.shape[1]),
            full(*d1.shape),
            full(*wt2.shape),
            full(*w2st.shape),
            full(1, b2s.shape[1]),
            full(*wc1t.shape),
            full(1, bc1_2d.shape[1]),
            full(*wc2t.shape),
            full(1, bc2_2d.shape[1]),
        ],
        out_specs=pl.BlockSpec((blk, num_class), lambda i: (i, 0)),
        out_shape=jax.ShapeDtypeStruct((n, num_class), jnp.float32),
    )(xs, wt1t, a1, d1, wt2, w2st, b2s, wc1t, bc1_2d, wc2t, bc2_2d)


def kernel(x, neighbor, W1, b1, Wt1, bt1, g1, be1, W2, b2, Wt2, bt2, g2, be2,
           Wc1, bc1, Wc2, bc2):
    return _kcat(x, neighbor, W1, b1, Wt1, bt1, g1, be1, W2, b2, Wt2, bt2,
                 g2, be2, Wc1, bc1, Wc2, bc2)


# speedup vs baseline: 3.8939x; 1.4016x over previous
import functools

import jax
import jax.numpy as jnp
from jax.experimental import pallas as pl
from jax.experimental.pallas import tpu as pltpu

_EPS_BN = 1e-5


def _fused_kernel(x_ref, nb_ref, wt1w_ref, a1w_ref, d1w_ref, wt2_ref,
                  w2st_ref, b2s_ref, wc1t_ref, bc1_ref, wc2t_ref, bc2_ref,
                  out_ref):
    x = x_ref[...]
    nb = nb_ref[...]

    s = jnp.sum(nb, axis=1)
    u = jnp.stack([x, s], axis=1)
    v = jnp.stack([s, x], axis=1)
    raw = jax.lax.dot_general(u, v, (((1,), (1,)), ((0,), (0,))),
                              preferred_element_type=jnp.float32)
    a_raw = jnp.abs(raw)
    q = a_raw * jax.lax.rsqrt(a_raw + 1e-30)
    sign_bit = jax.lax.bitcast_convert_type(raw, jnp.uint32) \
        & jnp.uint32(0x80000000)
    a_mat = jax.lax.bitcast_convert_type(
        jax.lax.bitcast_convert_type(q, jnp.uint32) | sign_bit, jnp.float32)

    ones_row = jnp.ones((x.shape[0], 1, x.shape[1]), jnp.float32)
    denom = jax.lax.dot_general(ones_row, q, (((2,), (1,)), ((0,), (0,))),
                                preferred_element_type=jnp.float32)[:, 0, :]
    inv_d = 1.0 / (denom + 1e-7)
    ones_m = jnp.ones((x.shape[0], nb.shape[1], 1), jnp.float32)
    inv_rep = jax.lax.dot_general(ones_m, inv_d[:, None, :],
                                  (((2,), (1,)), ((0,), (0,))),
                                  preferred_element_type=jnp.float32)

    e_x = jnp.einsum('nf,nfg->ng', x, a_mat,
                     preferred_element_type=jnp.float32) * inv_d
    e_nb = jnp.einsum('nmf,nfg->nmg', nb, a_mat,
                      preferred_element_type=jnp.float32) * inv_rep
    wt1w = wt1w_ref[...]
    a1w = a1w_ref[...][0]
    d1w = d1w_ref[...][0]
    t_x = jnp.dot(e_x, wt1w, preferred_element_type=jnp.float32)
    t_nb = jnp.einsum('nmg,gh->nmh', e_nb, wt1w,
                      preferred_element_type=jnp.float32)
    pre_x = a1w[None] * t_x + d1w[None]
    khop_w = pre_x / (1.0 + jnp.abs(pre_x))
    pre_nb = a1w[None, None] * t_nb + d1w[None, None]
    act_nb = pre_nb / (1.0 + jnp.abs(pre_nb))
    khop = khop_w.reshape(-1, 4, 16)
    snbr = jnp.sum(act_nb, axis=1).reshape(-1, 4, 16)

    raw2 = jnp.einsum('ncf,ncg->nfg', khop, snbr,
                      preferred_element_type=jnp.float32)
    raw2 = raw2 + jnp.transpose(raw2, (0, 2, 1))
    ar2 = jnp.abs(raw2)
    q2 = ar2 * jax.lax.rsqrt(ar2 + 1e-30)
    sb2 = jax.lax.bitcast_convert_type(raw2, jnp.uint32) \
        & jnp.uint32(0x80000000)
    a2_mat = jax.lax.bitcast_convert_type(
        jax.lax.bitcast_convert_type(q2, jnp.uint32) | sb2, jnp.float32)
    inv_d2 = 1.0 / (jnp.sum(q2, axis=1) + 1e-7)

    e2 = jnp.einsum('ncf,nfg->ncg', khop, a2_mat,
                    preferred_element_type=jnp.float32)
    e2 = e2 * inv_d2[:, None, :]

    v = jnp.einsum('ncg,gz->ncz', e2, wt2_ref[...],
                   preferred_element_type=jnp.float32)[:, :, 0]
    z = jnp.dot(v, w2st_ref[...],
                preferred_element_type=jnp.float32) + b2s_ref[...]
    flat = z / (1.0 + jnp.abs(z))

    h = jnp.dot(flat, wc1t_ref[...],
                preferred_element_type=jnp.float32) + bc1_ref[...]
    h = jnp.maximum(h, 0.0)
    out_ref[...] = jnp.dot(h, wc2t_ref[...],
                           preferred_element_type=jnp.float32) + bc2_ref[...]


@jax.jit
def _kcat(x, neighbor, W1, b1, Wt1, bt1, g1, be1, W2, b2, Wt2, bt2, g2, be2,
          Wc1, bc1, Wc2, bc2):
    n = x.shape[0]
    f = x.shape[2]
    num_class = Wc2.shape[0]

    xin = x.reshape(n, f)
    nb = neighbor.reshape(n, neighbor.shape[2], f)

    scale1 = g1 / jnp.sqrt(1.0 + _EPS_BN)
    a1 = (scale1 * W1[:, 0]).reshape(1, -1)
    swt1 = jnp.sum(Wt1, axis=1)
    d1 = scale1[:, None] * (b1[:, None] * swt1[None, :] + bt1[None, :]) \
        + be1[:, None]

    scale2 = g2 / jnp.sqrt(1.0 + _EPS_BN)
    swt2 = jnp.sum(Wt2[0])
    w2st = (scale2[:, None] * W2).T
    b2s = (scale2 * (b2 * swt2 + bt2[0]) + be2).reshape(1, -1)

    wt1w = jnp.tile(Wt1.T, (1, 4))
    a1w = jnp.repeat(scale1 * W1[:, 0], 16).reshape(1, 64)
    d1w = d1.reshape(1, 64)
    wt2 = Wt2.T
    wc1t = Wc1.T
    wc2t = Wc2.T
    bc1_2d = bc1.reshape(1, -1)
    bc2_2d = bc2.reshape(1, -1)

    for blk in (80, 40, 16, 8, 5, 4, 2, 1):
        if n % blk == 0:
            break

    full = lambda *shape: pl.BlockSpec(shape, lambda i: (0,) * len(shape))
    return pl.pallas_call(
        _fused_kernel,
        grid=(n // blk,),
        in_specs=[
            pl.BlockSpec((blk, f), lambda i: (i, 0)),
            pl.BlockSpec((blk, nb.shape[1], f), lambda i: (i, 0, 0)),
            full(*wt1w.shape),
            full(*a1w.shape),
            full(*d1w.shape),
            full(*wt2.shape),
            full(*w2st.shape),
            full(1, b2s.shape[1]),
            full(*wc1t.shape),
            full(1, bc1_2d.shape[1]),
            full(*wc2t.shape),
            full(1, bc2_2d.shape[1]),
        ],
        out_specs=pl.BlockSpec((blk, num_class), lambda i: (i, 0)),
        out_shape=jax.ShapeDtypeStruct((n, num_class), jnp.float32),
    )(xin, nb, wt1w, a1w, d1w, wt2, w2st, b2s, wc1t, bc1_2d, wc2t, bc2_2d)


def kernel(x, neighbor, W1, b1, Wt1, bt1, g1, be1, W2, b2, Wt2, bt2, g2, be2,
           Wc1, bc1, Wc2, bc2):
    return _kcat(x, neighbor, W1, b1, Wt1, bt1, g1, be1, W2, b2, Wt2, bt2,
                 g2, be2, Wc1, bc1, Wc2, bc2)
